# baseline (speedup 1.0000x reference)
import functools

import jax
import jax.numpy as jnp
from jax import lax
from jax.experimental import pallas as pl
from jax.experimental.pallas import tpu as pltpu
from jax.experimental.pallas import tpu_sc as plsc

_L = 16


@functools.lru_cache(maxsize=None)
def _build(Br, F, V, D, NC, NS):
    NW = NC * NS
    bpw = Br // NW
    rpw = bpw * F
    NG = bpw // _L
    mesh = plsc.VectorSubcoreMesh(core_axis_name="c", subcore_axis_name="s")

    @functools.partial(
        pl.kernel,
        mesh=mesh,
        compiler_params=pltpu.CompilerParams(
            use_tc_tiling_on_sc=True, needs_layout_passes=False
        ),
        out_type=jax.ShapeDtypeStruct((F, D, Br), jnp.float32),
        scratch_types=[
            pltpu.VMEM((rpw,), jnp.int32),
            [pltpu.VMEM((bpw,), jnp.int32)] * 2,
            [pltpu.VMEM((bpw,), jnp.int32)] * 2,
            [pltpu.VMEM((bpw, 2 * D), jnp.float32)] * 2,
            pltpu.VMEM((D, bpw), jnp.float32),
            [pltpu.SemaphoreType.DMA] * 2,
        ],
    )
    def emb(idx_hbm, table_hbm, out_hbm, idx_v, pids, cbs, bufs, obuf, gsems):
        wid = lax.axis_index("s") * NC + lax.axis_index("c")
        pltpu.sync_copy(idx_hbm.at[pl.ds(wid * rpw, rpw)], idx_v)

        lane = lax.iota(jnp.int32, _L)
        posg = [lane * F + g * _L * F for g in range(NG)]
        rowg = [lane + g * _L for g in range(NG)]

        def stage(f, s):
            for g in range(NG):
                raw = plsc.load_gather(idx_v, [posg[g] + f])
                pids[s][pl.ds(g * _L, _L)] = lax.shift_right_logical(raw, 1)
                cbs[s][pl.ds(g * _L, _L)] = lax.shift_left(
                    lax.bitwise_and(raw, 1), 6
                )

        def gather(s):
            return pltpu.async_copy(table_hbm.at[pids[s]], bufs[s], gsems[s])

        def extract(f, s):
            pltpu.make_async_copy(table_hbm.at[pids[s]], bufs[s], gsems[s]).wait()
            cbv = [cbs[s][pl.ds(g * _L, _L)] for g in range(NG)]
            for d in range(D):
                for g in range(NG):
                    vals = plsc.load_gather(bufs[s], [rowg[g], cbv[g] + d])
                    obuf[d, pl.ds(g * _L, _L)] = vals
            pltpu.sync_copy(obuf, out_hbm.at[f, :, pl.ds(wid * bpw, bpw)])

        stage(0, 0)
        gather(0)

        def body(t, carry):
            f0 = 2 * t
            stage(f0 + 1, 1)
            gather(1)
            extract(f0, 0)

            @pl.when(t < F // 2 - 1)
            def _():
                stage(f0 + 2, 0)
                gather(0)

            extract(f0 + 1, 1)
            return carry

        lax.fori_loop(0, F // 2, body, 0)

    return emb


def kernel(input, emb_weight):
    Br, F = input.shape
    V, D = emb_weight.shape
    info = plsc.get_sparse_core_info()
    NC, NS = info.num_cores, info.num_subcores
    idx = input.reshape(-1).astype(jnp.int32)
    table2 = emb_weight.reshape(V // 2, 2 * D)
    out_t = _build(Br, F, V, D, NC, NS)(idx, table2)
    return jnp.transpose(out_t, (2, 0, 1))


# speedup vs baseline: 1.1555x; 1.1555x over previous
import functools

import jax
import jax.numpy as jnp
from jax import lax
from jax.experimental import pallas as pl
from jax.experimental.pallas import tpu as pltpu
from jax.experimental.pallas import tpu_sc as plsc

_L = 16


@functools.lru_cache(maxsize=None)
def _build(Br, F, V, D, NC, NS):
    NW = NC * NS
    bpw = Br // NW
    rpw = bpw * F
    NG = bpw // _L
    NDG = D // _L
    mesh = plsc.VectorSubcoreMesh(core_axis_name="c", subcore_axis_name="s")

    @functools.partial(
        pl.kernel,
        mesh=mesh,
        compiler_params=pltpu.CompilerParams(
            use_tc_tiling_on_sc=True, needs_layout_passes=False
        ),
        out_type=jax.ShapeDtypeStruct((F, D, Br), jnp.float32),
        scratch_types=[
            pltpu.VMEM((rpw,), jnp.int32),
            [pltpu.VMEM((bpw,), jnp.int32)] * 2,
            [pltpu.VMEM((bpw,), jnp.int32)] * 2,
            [pltpu.VMEM((bpw, 2 * D), jnp.float32)] * 2,
            pltpu.VMEM((D, bpw), jnp.float32),
            [pltpu.SemaphoreType.DMA] * 2,
        ],
    )
    def emb(idx_hbm, table_hbm, out_hbm, idx_v, pids, cbs, bufs, obuf, gsems):
        wid = lax.axis_index("s") * NC + lax.axis_index("c")
        pltpu.sync_copy(idx_hbm.at[pl.ds(wid * rpw, rpw)], idx_v)

        lane = lax.iota(jnp.int32, _L)
        perm = [lax.rem(lane + dd, jnp.int32(_L)) for dd in range(_L)]

        def stage(f, s):
            def sg(g, c):
                raw = plsc.load_gather(idx_v, [(lane + g * _L) * F + f])
                pids[s][pl.ds(g * _L, _L)] = lax.shift_right_logical(raw, 1)
                cbs[s][pl.ds(g * _L, _L)] = lax.shift_left(
                    lax.bitwise_and(raw, 1), 6
                )
                return c

            lax.fori_loop(0, NG, sg, 0)

        def gather(s):
            return pltpu.async_copy(table_hbm.at[pids[s]], bufs[s], gsems[s])

        def extract(f, s):
            pltpu.make_async_copy(
                table_hbm.at[pids[s]], bufs[s], gsems[s]
            ).wait()

            def eg(g, c):
                rowv = lane + g * _L
                cbv = cbs[s][pl.ds(g * _L, _L)]
                for dg in range(NDG):
                    base = cbv + dg * _L
                    for dd in range(_L):
                        vals = plsc.load_gather(
                            bufs[s], [rowv, base + perm[dd]]
                        )
                        plsc.store_scatter(
                            obuf, [perm[dd] + dg * _L, rowv], vals
                        )
                return c

            lax.fori_loop(0, NG, eg, 0)
            pltpu.sync_copy(obuf, out_hbm.at[f, :, pl.ds(wid * bpw, bpw)])

        stage(0, 0)
        gather(0)

        def body(t, carry):
            f0 = 2 * t
            stage(f0 + 1, 1)
            gather(1)
            extract(f0, 0)

            @pl.when(t < F // 2 - 1)
            def _():
                stage(f0 + 2, 0)
                gather(0)

            extract(f0 + 1, 1)
            return carry

        lax.fori_loop(0, F // 2, body, 0)

    return emb


def kernel(input, emb_weight):
    Br, F = input.shape
    V, D = emb_weight.shape
    info = plsc.get_sparse_core_info()
    NC, NS = info.num_cores, info.num_subcores
    idx = input.reshape(-1).astype(jnp.int32)
    table2 = emb_weight.reshape(V // 2, 2 * D)
    out_t = _build(Br, F, V, D, NC, NS)(idx, table2)
    return jnp.transpose(out_t, (2, 0, 1))


# speedup vs baseline: 1.7952x; 1.5535x over previous
import functools

import jax
import jax.numpy as jnp
from jax import lax
from jax.experimental import pallas as pl
from jax.experimental.pallas import tpu as pltpu
from jax.experimental.pallas import tpu_sc as plsc

_L = 16


@functools.lru_cache(maxsize=None)
def _build(Br, F, V, D, NC, NS):
    NW = NC * NS
    bpw = Br // NW
    rpw = bpw * F
    HW = bpw // 4
    NHG = HW // _L
    NDG = D // _L
    mesh = plsc.VectorSubcoreMesh(core_axis_name="c", subcore_axis_name="s")

    @functools.partial(
        pl.kernel,
        mesh=mesh,
        compiler_params=pltpu.CompilerParams(
            use_tc_tiling_on_sc=True, needs_layout_passes=False
        ),
        out_type=jax.ShapeDtypeStruct((F, D, Br), jnp.float32),
        scratch_types=[
            pltpu.VMEM((rpw,), jnp.int32),
            [pltpu.VMEM((HW,), jnp.int32)] * 2,
            [pltpu.VMEM((HW,), jnp.int32)] * 2,
            [pltpu.VMEM((HW, 8, D), jnp.float32)] * 2,
            pltpu.VMEM((D, bpw), jnp.float32),
            [pltpu.SemaphoreType.DMA] * 2,
        ],
    )
    def emb(idx_hbm, table_hbm, out_hbm, idx_v, pids, r8s, bufs, obuf, gsems):
        wid = lax.axis_index("s") * NC + lax.axis_index("c")
        pltpu.sync_copy(idx_hbm.at[pl.ds(wid * rpw, rpw)], idx_v)

        lane = lax.iota(jnp.int32, _L)
        perm = [lax.rem(lane + dd, jnp.int32(_L)) for dd in range(_L)]

        def stage(f, q, s):
            def sg(g, c):
                pos = (lane + q * HW + g * _L) * F + f
                raw = plsc.load_gather(idx_v, [pos])
                pids[s][pl.ds(g * _L, _L)] = lax.shift_right_logical(raw, 3)
                r8s[s][pl.ds(g * _L, _L)] = lax.bitwise_and(raw, 7)
                return c

            lax.fori_loop(0, NHG, sg, 0)

        def gather(s):
            def gj(g, c):
                vec = pids[s][pl.ds(g * _L, _L)]
                for kk in range(_L):
                    pltpu.async_copy(
                        table_hbm.at[pl.ds(vec[kk], 1)],
                        bufs[s].at[pl.ds(g * _L + kk, 1)],
                        gsems[s],
                    )
                return c

            lax.fori_loop(0, NHG, gj, 0)

        def transpose(q, s):
            pltpu.make_async_copy(
                table_hbm.at[pl.ds(0, HW)], bufs[s], gsems[s]
            ).wait()

            def eg(g, c):
                slotv = lane + g * _L
                r8v = r8s[s][pl.ds(g * _L, _L)]
                colb = slotv + q * HW
                for dg in range(NDG):
                    for dd in range(_L):
                        dv = perm[dd] + dg * _L
                        vals = plsc.load_gather(bufs[s], [slotv, r8v, dv])
                        plsc.store_scatter(obuf, [dv, colb], vals)
                return c

            lax.fori_loop(0, NHG, eg, 0)

        stage(0, 0, 0)
        gather(0)

        def body(f, carry):
            stage(f, 1, 1)
            gather(1)
            transpose(0, 0)
            stage(f, 2, 0)
            gather(0)
            transpose(1, 1)
            stage(f, 3, 1)
            gather(1)
            transpose(2, 0)

            @pl.when(f < F - 1)
            def _():
                stage(f + 1, 0, 0)
                gather(0)

            transpose(3, 1)
            pltpu.sync_copy(obuf, out_hbm.at[f, :, pl.ds(wid * bpw, bpw)])
            return carry

        lax.fori_loop(0, F, body, 0)

    return emb


def kernel(input, emb_weight):
    Br, F = input.shape
    V, D = emb_weight.shape
    info = plsc.get_sparse_core_info()
    NC, NS = info.num_cores, info.num_subcores
    idx = input.reshape(-1).astype(jnp.int32)
    table3 = emb_weight.reshape(V // 8, 8, D)
    out_t = _build(Br, F, V, D, NC, NS)(idx, table3)
    return jnp.transpose(out_t, (2, 0, 1))
